# Initial kernel scaffold; baseline (speedup 1.0000x reference)
import functools

import jax
import jax.numpy as jnp
from jax import lax
from jax.experimental import pallas as pl
from jax.experimental.pallas import tpu as pltpu
from jax.experimental.pallas import tpu_sc as plsc

POINT_INFLUENCE = 0.04 * 30.0
INV_SIGMA = 1.0 / POINT_INFLUENCE

NC = 2
NS = 16
NW = NC * NS
TPC = 10


def _sc_gather(xyzT_flat, idx_t, n, tot):
    ni = tot // TPC
    sb = 4000
    assert ni % sb == 0 and sb % 16 == 0 and n % 8 == 0 and ni % 8 == 0

    mesh = plsc.VectorSubcoreMesh(core_axis_name="c", subcore_axis_name="s")

    @functools.partial(
        pl.kernel,
        out_type=jax.ShapeDtypeStruct((3 * tot,), jnp.float32),
        mesh=mesh,
        scratch_types=[
            pltpu.VMEM((n,), jnp.float32),
            pltpu.VMEM((sb,), jnp.int32),
            pltpu.VMEM((sb,), jnp.float32),
        ],
    )
    def sc_kernel(xyz_hbm, idx_hbm, out_hbm, plane_v, idx_v, val_v):
        wid = lax.axis_index("s") * NC + lax.axis_index("c")
        coord = wid // TPC
        chunk = wid % TPC

        @pl.when(coord < 3)
        def _():
            pltpu.sync_copy(xyz_hbm.at[pl.ds(coord * n, n)], plane_v)
            ibase = chunk * ni

            def slab(si, _):
                off = ibase + si * sb
                pltpu.sync_copy(idx_hbm.at[pl.ds(off, sb)], idx_v)

                def step(t, _):
                    iv = idx_v[pl.ds(t * 16, 16)]
                    val_v[pl.ds(t * 16, 16)] = plsc.load_gather(plane_v, [iv])
                    return 0

                lax.fori_loop(0, sb // 16, step, 0, unroll=8)
                pltpu.sync_copy(val_v, out_hbm.at[pl.ds(coord * tot + off, sb)])
                return 0

            lax.fori_loop(0, ni // sb, slab, 0)

    return sc_kernel(xyzT_flat, idx_t)


def _tc_conv(gath, centT, kpT, w45, q, s, ql):
    nblk = q // ql

    def body(gath_ref, cent_ref, kp_ref, w_ref, out_ref):
        xg = gath_ref[0]
        yg = gath_ref[1]
        zg = gath_ref[2]
        cx = cent_ref[0:1, :]
        cy = cent_ref[1:2, :]
        cz = cent_ref[2:3, :]
        relx = xg - cx
        rely = yg - cy
        relz = zg - cz
        r2 = relx * relx + rely * rely + relz * relz
        rows = []
        for k in range(15):
            kx = kp_ref[0, k]
            ky = kp_ref[1, k]
            kz = kp_ref[2, k]
            kn2 = kx * kx + ky * ky + kz * kz
            d2 = r2 - 2.0 * (relx * kx + rely * ky + relz * kz) + kn2
            d2 = jnp.maximum(d2, 0.0)
            w = jnp.maximum(1.0 - jnp.sqrt(d2) * INV_SIGMA, 0.0)
            rows.append(jnp.sum(w * xg, axis=0, keepdims=True))
            rows.append(jnp.sum(w * yg, axis=0, keepdims=True))
            rows.append(jnp.sum(w * zg, axis=0, keepdims=True))
        h = jnp.concatenate(rows, axis=0)
        out_ref[...] = lax.dot_general(
            h, w_ref[...], (((0,), (0,)), ((), ())),
            preferred_element_type=jnp.float32)

    return pl.pallas_call(
        body,
        grid=(nblk,),
        in_specs=[
            pl.BlockSpec((3, s, ql), lambda i: (0, 0, i)),
            pl.BlockSpec((3, ql), lambda i: (0, i)),
            pl.BlockSpec(memory_space=pltpu.SMEM),
            pl.BlockSpec((45, 64), lambda i: (0, 0)),
        ],
        out_specs=pl.BlockSpec((ql, 64), lambda i: (i, 0)),
        out_shape=jax.ShapeDtypeStruct((q, 64), jnp.float32),
    )(gath, centT, kpT, w45)


def _tc_bn(x, gamma1, beta1):
    def body(x_ref, g_ref, b_ref, o_ref):
        xv = x_ref[...]
        mean = jnp.mean(xv, axis=0, keepdims=True)
        xc = xv - mean
        var = jnp.mean(xc * xc, axis=0, keepdims=True)
        y = xc * lax.rsqrt(var + 1e-5) * g_ref[...] + b_ref[...]
        o_ref[...] = jnp.where(y >= 0, y, 0.2 * y)

    return pl.pallas_call(
        body,
        out_shape=jax.ShapeDtypeStruct(x.shape, jnp.float32),
    )(x, gamma1, beta1)


def kernel(xyz, centors, idx, K_points, weight, gamma, beta):
    b, n, _ = xyz.shape
    num_group = centors.shape[1]
    q = b * num_group
    s = idx.shape[0] // q
    tot = q * s

    xyzT_flat = xyz.reshape(n, 3).T.reshape(-1)
    idx_t = idx.reshape(q, s).T.reshape(-1)
    centT = centors.reshape(q, 3).T
    kpT = K_points.T
    w45 = weight[:, 3:6, :].reshape(45, 64)

    gathered = _sc_gather(xyzT_flat, idx_t, n, tot)
    gath = gathered.reshape(3, s, q)
    out_pre = _tc_conv(gath, centT, kpT, w45, q, s, ql=1000)
    out = _tc_bn(out_pre, gamma.reshape(1, 64), beta.reshape(1, 64))
    return out.reshape(b, num_group, 64)


# speedup vs baseline: 31.5646x; 31.5646x over previous
import functools

import jax
import jax.numpy as jnp
from jax import lax
from jax.experimental import pallas as pl
from jax.experimental.pallas import tpu as pltpu
from jax.experimental.pallas import tpu_sc as plsc

POINT_INFLUENCE = 0.04 * 30.0
INV_SIGMA = 1.0 / POINT_INFLUENCE

NC = 2
NS = 16
NW = NC * NS


def _sc_gather(xyzT_flat, idx_t, n, q, s, qpad):
    ntask = 3 * s
    tpw = ntask // NW
    assert ntask % NW == 0 and q % 16 == 0 and n % 8 == 0 and qpad % 8 == 0

    mesh = plsc.VectorSubcoreMesh(core_axis_name="c", subcore_axis_name="s")

    @functools.partial(
        pl.kernel,
        out_type=jax.ShapeDtypeStruct((3 * s * qpad,), jnp.float32),
        mesh=mesh,
        scratch_types=[
            pltpu.VMEM((n,), jnp.float32),
            pltpu.VMEM((q,), jnp.int32),
            pltpu.VMEM((q,), jnp.float32),
        ],
        compiler_params=pltpu.CompilerParams(needs_layout_passes=False),
    )
    def sc_kernel(xyz_hbm, idx_hbm, out_hbm, plane_v, idx_v, val_v):
        wid = lax.axis_index("s") * NC + lax.axis_index("c")

        def task(t, loaded):
            rt = wid * tpw + t
            coord = rt // s
            j = rt % s

            @pl.when(coord != loaded)
            def _():
                pltpu.sync_copy(xyz_hbm.at[pl.ds(coord * n, n)], plane_v)

            pltpu.sync_copy(idx_hbm.at[pl.ds(j * q, q)], idx_v)

            def step(u, _):
                iv = idx_v[pl.ds(u * 16, 16)]
                val_v[pl.ds(u * 16, 16)] = plsc.load_gather(plane_v, [iv])
                return 0

            lax.fori_loop(0, q // 16, step, 0, unroll=8)
            pltpu.sync_copy(val_v, out_hbm.at[pl.ds(rt * qpad, q)])
            return coord

        lax.fori_loop(0, tpw, task, -1)

    return sc_kernel(xyzT_flat, idx_t)


def _tc_conv(gath, centT, kpT, w45, qpad, s, ql):
    nblk = qpad // ql

    def body(gath_ref, cent_ref, kp_ref, w_ref, out_ref):
        xg = gath_ref[0]
        yg = gath_ref[1]
        zg = gath_ref[2]
        cx = cent_ref[0:1, :]
        cy = cent_ref[1:2, :]
        cz = cent_ref[2:3, :]
        relx = xg - cx
        rely = yg - cy
        relz = zg - cz
        r2 = relx * relx + rely * rely + relz * relz
        rows = []
        for k in range(15):
            kx = kp_ref[0, k]
            ky = kp_ref[1, k]
            kz = kp_ref[2, k]
            kn2 = kx * kx + ky * ky + kz * kz
            d2 = r2 - 2.0 * (relx * kx + rely * ky + relz * kz) + kn2
            d2 = jnp.maximum(d2, 0.0)
            w = jnp.maximum(1.0 - jnp.sqrt(d2) * INV_SIGMA, 0.0)
            rows.append(jnp.sum(w * xg, axis=0, keepdims=True))
            rows.append(jnp.sum(w * yg, axis=0, keepdims=True))
            rows.append(jnp.sum(w * zg, axis=0, keepdims=True))
        h = jnp.concatenate(rows, axis=0)
        out_ref[...] = lax.dot_general(
            h, w_ref[...], (((0,), (0,)), ((), ())),
            preferred_element_type=jnp.float32)

    return pl.pallas_call(
        body,
        grid=(nblk,),
        in_specs=[
            pl.BlockSpec((3, s, ql), lambda i: (0, 0, i)),
            pl.BlockSpec((3, ql), lambda i: (0, i)),
            pl.BlockSpec(memory_space=pltpu.SMEM),
            pl.BlockSpec((45, 64), lambda i: (0, 0)),
        ],
        out_specs=pl.BlockSpec((ql, 64), lambda i: (i, 0)),
        out_shape=jax.ShapeDtypeStruct((qpad, 64), jnp.float32),
    )(gath, centT, kpT, w45)


def _tc_bn(x, gamma1, beta1, q):
    qpad = x.shape[0]

    def body(x_ref, g_ref, b_ref, o_ref):
        xv = x_ref[...]
        rid = lax.broadcasted_iota(jnp.int32, xv.shape, 0)
        m = rid < q
        xz = jnp.where(m, xv, 0.0)
        mean = jnp.sum(xz, axis=0, keepdims=True) * (1.0 / q)
        xc = xv - mean
        var = jnp.sum(jnp.where(m, xc * xc, 0.0), axis=0, keepdims=True) * (1.0 / q)
        y = xc * lax.rsqrt(var + 1e-5) * g_ref[...] + b_ref[...]
        o_ref[...] = jnp.where(y >= 0, y, 0.2 * y)[:q]

    return pl.pallas_call(
        body,
        out_shape=jax.ShapeDtypeStruct((q, 64), jnp.float32),
    )(x, gamma1, beta1)


def kernel(xyz, centors, idx, K_points, weight, gamma, beta):
    b, n, _ = xyz.shape
    num_group = centors.shape[1]
    q = b * num_group
    s = idx.shape[0] // q
    qpad = ((q + 1023) // 1024) * 1024

    xyzT_flat = xyz.reshape(n, 3).T.reshape(-1)
    idx_t = idx.reshape(q, s).T.reshape(-1)
    centT = jnp.pad(centors.reshape(q, 3).T, ((0, 0), (0, qpad - q)))
    kpT = K_points.T
    w45 = weight[:, 3:6, :].reshape(45, 64)

    gathered = _sc_gather(xyzT_flat, idx_t, n, q, s, qpad)
    gath = gathered.reshape(3, s, qpad)
    out_pre = _tc_conv(gath, centT, kpT, w45, qpad, s, ql=1024)
    out = _tc_bn(out_pre, gamma.reshape(1, 64), beta.reshape(1, 64), q)
    return out.reshape(b, num_group, 64)


# speedup vs baseline: 33.5362x; 1.0625x over previous
import functools

import jax
import jax.numpy as jnp
from jax import lax
from jax.experimental import pallas as pl
from jax.experimental.pallas import tpu as pltpu
from jax.experimental.pallas import tpu_sc as plsc

POINT_INFLUENCE = 0.04 * 30.0
INV_SIGMA = 1.0 / POINT_INFLUENCE

NC = 2
NS = 16
NW = NC * NS
H1 = 5008


def _sc_gather(xyzT_flat, idx_t, n, q, s, qpad):
    h2 = q - H1
    assert h2 % 16 == 0 and H1 % 16 == 0 and n % 8 == 0 and qpad % 8 == 0
    max_rows = (s + 9) // 10
    nslot = 2 * max_rows

    mesh = plsc.VectorSubcoreMesh(core_axis_name="c", subcore_axis_name="s")

    @functools.partial(
        pl.kernel,
        out_type=jax.ShapeDtypeStruct((3 * s * qpad,), jnp.float32),
        mesh=mesh,
        scratch_types=[
            pltpu.VMEM((n,), jnp.float32),
            pltpu.VMEM((H1,), jnp.int32),
            pltpu.VMEM((h2,), jnp.int32),
            pltpu.VMEM((H1,), jnp.float32),
            pltpu.VMEM((h2,), jnp.float32),
            pltpu.SemaphoreType.DMA,
            pltpu.SemaphoreType.DMA,
            pltpu.SemaphoreType.DMA,
            pltpu.SemaphoreType.DMA,
            pltpu.SemaphoreType.DMA,
        ],
        compiler_params=pltpu.CompilerParams(needs_layout_passes=False),
    )
    def sc_kernel(xyz_hbm, idx_hbm, out_hbm, plane_v, idx_v0, idx_v1,
                  val_v0, val_v1, sem_p, sem_i0, sem_i1, sem_o0, sem_o1):
        wid = lax.axis_index("s") * NC + lax.axis_index("c")
        coord = jnp.where(wid < 11, 0, jnp.where(wid < 22, 1, 2))
        base = wid - jnp.where(wid < 11, 0, jnp.where(wid < 22, 11, 22))
        tiles = jnp.where(wid < 22, 11, 10)
        r0 = base * s // tiles
        r1 = (base + 1) * s // tiles
        nh = 2 * (r1 - r0)

        sem_i = (sem_i0, sem_i1)
        sem_o = (sem_o0, sem_o1)
        idx_bufs = (idx_v0, idx_v1)
        val_bufs = (val_v0, val_v1)

        def task_offs(h):
            j = r0 + h // 2
            half = h % 2
            ioff = j * q + half * H1
            ooff = (coord * s + j) * qpad + half * H1
            ln = H1 if half == 0 else h2
            return ioff, ooff, ln

        def idx_cp(h):
            ioff, _, ln = task_offs(h)
            return pltpu.make_async_copy(
                idx_hbm.at[pl.ds(ioff, ln)], idx_bufs[h % 2], sem_i[h % 2])

        def out_cp(h):
            _, ooff, ln = task_offs(h)
            return pltpu.make_async_copy(
                val_bufs[h % 2], out_hbm.at[pl.ds(ooff, ln)], sem_o[h % 2])

        plane_cp = pltpu.make_async_copy(
            xyz_hbm.at[pl.ds(coord * n, n)], plane_v, sem_p)
        plane_cp.start()
        idx_cp(0).start()
        idx_cp(1).start()
        plane_cp.wait()

        for h in range(nslot):
            b = h % 2
            if h >= 2:
                @pl.when(h - 2 < nh)
                def _(h=h):
                    out_cp(h - 2).wait()

            @pl.when(h < nh)
            def _(h=h, b=b):
                idx_cp(h).wait()
                _, _, ln = task_offs(h)

                def step(u, _):
                    iv = idx_bufs[b][pl.ds(u * 16, 16)]
                    val_bufs[b][pl.ds(u * 16, 16)] = plsc.load_gather(
                        plane_v, [iv])
                    return 0

                lax.fori_loop(0, ln // 16, step, 0, unroll=8)
                out_cp(h).start()

            if h + 2 < nslot:
                @pl.when(h + 2 < nh)
                def _(h=h):
                    idx_cp(h + 2).start()

        for h in range(nslot - 2, nslot):
            @pl.when(h < nh)
            def _(h=h):
                out_cp(h).wait()

    return sc_kernel(xyzT_flat, idx_t)


def _tc_conv(gath, centT, kpT, w45, qpad, s, ql):
    nblk = qpad // ql

    def body(gath_ref, cent_ref, kp_ref, w_ref, out_ref):
        xg = gath_ref[0]
        yg = gath_ref[1]
        zg = gath_ref[2]
        cx = cent_ref[0:1, :]
        cy = cent_ref[1:2, :]
        cz = cent_ref[2:3, :]
        relx = xg - cx
        rely = yg - cy
        relz = zg - cz
        rows = []
        for k in range(15):
            dx = relx - kp_ref[0, k]
            dy = rely - kp_ref[1, k]
            dz = relz - kp_ref[2, k]
            d2 = dx * dx + dy * dy + dz * dz
            w = jnp.maximum(1.0 - jnp.sqrt(d2) * INV_SIGMA, 0.0)
            rows.append(jnp.sum(w * xg, axis=0, keepdims=True))
            rows.append(jnp.sum(w * yg, axis=0, keepdims=True))
            rows.append(jnp.sum(w * zg, axis=0, keepdims=True))
        h = jnp.concatenate(rows, axis=0)
        out_ref[...] = lax.dot_general(
            h, w_ref[...], (((0,), (0,)), ((), ())),
            preferred_element_type=jnp.float32)

    return pl.pallas_call(
        body,
        grid=(nblk,),
        in_specs=[
            pl.BlockSpec((3, s, ql), lambda i: (0, 0, i)),
            pl.BlockSpec((3, ql), lambda i: (0, i)),
            pl.BlockSpec(memory_space=pltpu.SMEM),
            pl.BlockSpec((45, 64), lambda i: (0, 0)),
        ],
        out_specs=pl.BlockSpec((ql, 64), lambda i: (i, 0)),
        out_shape=jax.ShapeDtypeStruct((qpad, 64), jnp.float32),
    )(gath, centT, kpT, w45)


def _tc_bn(x, gamma1, beta1, q):
    def body(x_ref, g_ref, b_ref, o_ref):
        xv = x_ref[...]
        rid = lax.broadcasted_iota(jnp.int32, xv.shape, 0)
        m = rid < q
        xz = jnp.where(m, xv, 0.0)
        mean = jnp.sum(xz, axis=0, keepdims=True) * (1.0 / q)
        xc = xv - mean
        var = jnp.sum(jnp.where(m, xc * xc, 0.0), axis=0, keepdims=True) * (1.0 / q)
        y = xc * lax.rsqrt(var + 1e-5) * g_ref[...] + b_ref[...]
        o_ref[...] = jnp.where(y >= 0, y, 0.2 * y)[:q]

    return pl.pallas_call(
        body,
        out_shape=jax.ShapeDtypeStruct((q, 64), jnp.float32),
    )(x, gamma1, beta1)


def kernel(xyz, centors, idx, K_points, weight, gamma, beta):
    b, n, _ = xyz.shape
    num_group = centors.shape[1]
    q = b * num_group
    s = idx.shape[0] // q
    qpad = ((q + 1023) // 1024) * 1024

    xyzT_flat = xyz.reshape(n, 3).T.reshape(-1)
    idx_t = idx.reshape(q, s).T.reshape(-1)
    centT = jnp.pad(centors.reshape(q, 3).T, ((0, 0), (0, qpad - q)))
    kpT = K_points.T
    w45 = weight[:, 3:6, :].reshape(45, 64)

    gathered = _sc_gather(xyzT_flat, idx_t, n, q, s, qpad)
    gath = gathered.reshape(3, s, qpad)
    out_pre = _tc_conv(gath, centT, kpT, w45, qpad, s, ql=1024)
    out = _tc_bn(out_pre, gamma.reshape(1, 64), beta.reshape(1, 64), q)
    return out.reshape(b, num_group, 64)


# speedup vs baseline: 42.1339x; 1.2564x over previous
import functools

import jax
import jax.numpy as jnp
from jax import lax
from jax.experimental import pallas as pl
from jax.experimental.pallas import tpu as pltpu
from jax.experimental.pallas import tpu_sc as plsc

POINT_INFLUENCE = 0.04 * 30.0
INV_SIGMA = 1.0 / POINT_INFLUENCE

NC = 2
NS = 16
NW = NC * NS
H1 = 5008


def _sc_gather(xyzT_flat, idx_t, n, q, s, qpad):
    h2 = q - H1
    assert h2 % 16 == 0 and H1 % 16 == 0 and n % 8 == 0 and qpad % 8 == 0
    max_rows = (s + 9) // 10
    nslot = 2 * max_rows

    mesh = plsc.VectorSubcoreMesh(core_axis_name="c", subcore_axis_name="s")

    @functools.partial(
        pl.kernel,
        out_type=jax.ShapeDtypeStruct((3 * s * qpad,), jnp.float32),
        mesh=mesh,
        scratch_types=[
            pltpu.VMEM((n,), jnp.float32),
            pltpu.VMEM((H1,), jnp.int32),
            pltpu.VMEM((h2,), jnp.int32),
            pltpu.VMEM((H1,), jnp.float32),
            pltpu.VMEM((h2,), jnp.float32),
            pltpu.SemaphoreType.DMA,
            pltpu.SemaphoreType.DMA,
            pltpu.SemaphoreType.DMA,
            pltpu.SemaphoreType.DMA,
            pltpu.SemaphoreType.DMA,
        ],
        compiler_params=pltpu.CompilerParams(needs_layout_passes=False),
    )
    def sc_kernel(xyz_hbm, idx_hbm, out_hbm, plane_v, idx_v0, idx_v1,
                  val_v0, val_v1, sem_p, sem_i0, sem_i1, sem_o0, sem_o1):
        wid = lax.axis_index("s") * NC + lax.axis_index("c")
        coord = jnp.where(wid < 11, 0, jnp.where(wid < 22, 1, 2))
        base = wid - jnp.where(wid < 11, 0, jnp.where(wid < 22, 11, 22))
        tiles = jnp.where(wid < 22, 11, 10)
        r0 = base * s // tiles
        r1 = (base + 1) * s // tiles
        nh = 2 * (r1 - r0)

        sem_i = (sem_i0, sem_i1)
        sem_o = (sem_o0, sem_o1)
        idx_bufs = (idx_v0, idx_v1)
        val_bufs = (val_v0, val_v1)

        def task_offs(h):
            j = r0 + h // 2
            half = h % 2
            ioff = j * q + half * H1
            ooff = (coord * s + j) * qpad + half * H1
            ln = H1 if half == 0 else h2
            return ioff, ooff, ln

        def idx_cp(h):
            ioff, _, ln = task_offs(h)
            return pltpu.make_async_copy(
                idx_hbm.at[pl.ds(ioff, ln)], idx_bufs[h % 2], sem_i[h % 2])

        def out_cp(h):
            _, ooff, ln = task_offs(h)
            return pltpu.make_async_copy(
                val_bufs[h % 2], out_hbm.at[pl.ds(ooff, ln)], sem_o[h % 2])

        plane_cp = pltpu.make_async_copy(
            xyz_hbm.at[pl.ds(coord * n, n)], plane_v, sem_p)
        plane_cp.start()
        idx_cp(0).start()
        idx_cp(1).start()
        plane_cp.wait()

        for h in range(nslot):
            b = h % 2
            if h >= 2:
                @pl.when(h - 2 < nh)
                def _(h=h):
                    out_cp(h - 2).wait()

            @pl.when(h < nh)
            def _(h=h, b=b):
                idx_cp(h).wait()
                _, _, ln = task_offs(h)

                @plsc.parallel_loop(0, ln, 16, unroll=8)
                def _(u):
                    iv = idx_bufs[b][pl.ds(u, 16)]
                    val_bufs[b][pl.ds(u, 16)] = plsc.load_gather(
                        plane_v, [iv])
                out_cp(h).start()

            if h + 2 < nslot:
                @pl.when(h + 2 < nh)
                def _(h=h):
                    idx_cp(h + 2).start()

        for h in range(nslot - 2, nslot):
            @pl.when(h < nh)
            def _(h=h):
                out_cp(h).wait()

    return sc_kernel(xyzT_flat, idx_t)


def _tc_conv(gath, centT, kpT, w45, qpad, s, ql):
    nblk = qpad // ql

    def body(gath_ref, cent_ref, kp_ref, w_ref, out_ref):
        xg = gath_ref[0]
        yg = gath_ref[1]
        zg = gath_ref[2]
        cx = cent_ref[0:1, :]
        cy = cent_ref[1:2, :]
        cz = cent_ref[2:3, :]
        relx = xg - cx
        rely = yg - cy
        relz = zg - cz
        rows = []
        for k in range(15):
            dx = relx - kp_ref[0, k]
            dy = rely - kp_ref[1, k]
            dz = relz - kp_ref[2, k]
            d2 = dx * dx + dy * dy + dz * dz
            w = jnp.maximum(1.0 - jnp.sqrt(d2) * INV_SIGMA, 0.0)
            rows.append(jnp.sum(w * xg, axis=0, keepdims=True))
            rows.append(jnp.sum(w * yg, axis=0, keepdims=True))
            rows.append(jnp.sum(w * zg, axis=0, keepdims=True))
        h = jnp.concatenate(rows, axis=0)
        out_ref[...] = lax.dot_general(
            h, w_ref[...], (((0,), (0,)), ((), ())),
            preferred_element_type=jnp.float32)

    return pl.pallas_call(
        body,
        grid=(nblk,),
        in_specs=[
            pl.BlockSpec((3, s, ql), lambda i: (0, 0, i)),
            pl.BlockSpec((3, ql), lambda i: (0, i)),
            pl.BlockSpec(memory_space=pltpu.SMEM),
            pl.BlockSpec((45, 64), lambda i: (0, 0)),
        ],
        out_specs=pl.BlockSpec((ql, 64), lambda i: (i, 0)),
        out_shape=jax.ShapeDtypeStruct((qpad, 64), jnp.float32),
    )(gath, centT, kpT, w45)


def _tc_bn(x, gamma1, beta1, q):
    def body(x_ref, g_ref, b_ref, o_ref):
        xv = x_ref[...]
        rid = lax.broadcasted_iota(jnp.int32, xv.shape, 0)
        m = rid < q
        xz = jnp.where(m, xv, 0.0)
        mean = jnp.sum(xz, axis=0, keepdims=True) * (1.0 / q)
        xc = xv - mean
        var = jnp.sum(jnp.where(m, xc * xc, 0.0), axis=0, keepdims=True) * (1.0 / q)
        y = xc * lax.rsqrt(var + 1e-5) * g_ref[...] + b_ref[...]
        o_ref[...] = jnp.where(y >= 0, y, 0.2 * y)[:q]

    return pl.pallas_call(
        body,
        out_shape=jax.ShapeDtypeStruct((q, 64), jnp.float32),
    )(x, gamma1, beta1)


def kernel(xyz, centors, idx, K_points, weight, gamma, beta):
    b, n, _ = xyz.shape
    num_group = centors.shape[1]
    q = b * num_group
    s = idx.shape[0] // q
    qpad = ((q + 1023) // 1024) * 1024

    xyzT_flat = xyz.reshape(n, 3).T.reshape(-1)
    idx_t = idx.reshape(q, s).T.reshape(-1)
    centT = jnp.pad(centors.reshape(q, 3).T, ((0, 0), (0, qpad - q)))
    kpT = K_points.T
    w45 = weight[:, 3:6, :].reshape(45, 64)

    gathered = _sc_gather(xyzT_flat, idx_t, n, q, s, qpad)
    gath = gathered.reshape(3, s, qpad)
    out_pre = _tc_conv(gath, centT, kpT, w45, qpad, s, ql=1024)
    out = _tc_bn(out_pre, gamma.reshape(1, 64), beta.reshape(1, 64), q)
    return out.reshape(b, num_group, 64)


# speedup vs baseline: 43.9868x; 1.0440x over previous
import functools

import jax
import jax.numpy as jnp
from jax import lax
from jax.experimental import pallas as pl
from jax.experimental.pallas import tpu as pltpu
from jax.experimental.pallas import tpu_sc as plsc

POINT_INFLUENCE = 0.04 * 30.0
INV_SIGMA = 1.0 / POINT_INFLUENCE

NC = 2
NS = 16
NW = NC * NS
H1 = 5008


def _sc_gather(xyzT_flat, idx_t, n, q, s, qpad):
    h2 = q - H1
    assert h2 % 16 == 0 and H1 % 16 == 0 and n % 8 == 0 and qpad % 8 == 0
    max_rows = (s + 9) // 10
    nslot = 2 * max_rows

    mesh = plsc.VectorSubcoreMesh(core_axis_name="c", subcore_axis_name="s")

    @functools.partial(
        pl.kernel,
        out_type=jax.ShapeDtypeStruct((3 * s * qpad,), jnp.float32),
        mesh=mesh,
        scratch_types=[
            pltpu.VMEM((n,), jnp.float32),
            pltpu.VMEM((H1,), jnp.int32),
            pltpu.VMEM((h2,), jnp.int32),
            pltpu.VMEM((H1,), jnp.float32),
            pltpu.VMEM((h2,), jnp.float32),
            pltpu.SemaphoreType.DMA,
            pltpu.SemaphoreType.DMA,
            pltpu.SemaphoreType.DMA,
            pltpu.SemaphoreType.DMA,
            pltpu.SemaphoreType.DMA,
        ],
        compiler_params=pltpu.CompilerParams(needs_layout_passes=False),
    )
    def sc_kernel(xyz_hbm, idx_hbm, out_hbm, plane_v, idx_v0, idx_v1,
                  val_v0, val_v1, sem_p, sem_i0, sem_i1, sem_o0, sem_o1):
        wid = lax.axis_index("s") * NC + lax.axis_index("c")
        coord = jnp.where(wid < 11, 0, jnp.where(wid < 22, 1, 2))
        base = wid - jnp.where(wid < 11, 0, jnp.where(wid < 22, 11, 22))
        tiles = jnp.where(wid < 22, 11, 10)
        r0 = base * s // tiles
        r1 = (base + 1) * s // tiles
        nh = 2 * (r1 - r0)

        sem_i = (sem_i0, sem_i1)
        sem_o = (sem_o0, sem_o1)
        idx_bufs = (idx_v0, idx_v1)
        val_bufs = (val_v0, val_v1)

        def task_offs(h):
            j = r0 + h // 2
            half = h % 2
            ioff = j * q + half * H1
            ooff = (coord * s + j) * qpad + half * H1
            ln = H1 if half == 0 else h2
            return ioff, ooff, ln

        def idx_cp(h):
            ioff, _, ln = task_offs(h)
            return pltpu.make_async_copy(
                idx_hbm.at[pl.ds(ioff, ln)], idx_bufs[h % 2], sem_i[h % 2])

        def out_cp(h):
            _, ooff, ln = task_offs(h)
            return pltpu.make_async_copy(
                val_bufs[h % 2], out_hbm.at[pl.ds(ooff, ln)], sem_o[h % 2])

        plane_cp = pltpu.make_async_copy(
            xyz_hbm.at[pl.ds(coord * n, n)], plane_v, sem_p)
        plane_cp.start()
        idx_cp(0).start()
        idx_cp(1).start()
        plane_cp.wait()

        for h in range(nslot):
            b = h % 2
            if h >= 2:
                @pl.when(h - 2 < nh)
                def _(h=h):
                    out_cp(h - 2).wait()

            @pl.when(h < nh)
            def _(h=h, b=b):
                idx_cp(h).wait()
                _, _, ln = task_offs(h)

                @plsc.parallel_loop(0, ln, 16, unroll=8)
                def _(u):
                    iv = idx_bufs[b][pl.ds(u, 16)]
                    val_bufs[b][pl.ds(u, 16)] = plsc.load_gather(
                        plane_v, [iv])
                out_cp(h).start()

            if h + 2 < nslot:
                @pl.when(h + 2 < nh)
                def _(h=h):
                    idx_cp(h + 2).start()

        for h in range(nslot - 2, nslot):
            @pl.when(h < nh)
            def _(h=h):
                out_cp(h).wait()

    return sc_kernel(xyzT_flat, idx_t)


def _tc_conv(gath, centT, kpT, w45, qpad, s, ql):
    nblk = qpad // ql

    def body(gath_ref, cent_ref, kp_ref, w_ref, out_ref):
        xg = gath_ref[0]
        yg = gath_ref[1]
        zg = gath_ref[2]
        cx = cent_ref[0:1, :]
        cy = cent_ref[1:2, :]
        cz = cent_ref[2:3, :]
        relx = xg - cx
        rely = yg - cy
        relz = zg - cz
        rows = []
        for k in range(15):
            dx = relx - kp_ref[0, k]
            dy = rely - kp_ref[1, k]
            dz = relz - kp_ref[2, k]
            d2 = jnp.maximum(dx * dx + dy * dy + dz * dz, 1e-24)
            w = jnp.maximum(1.0 - (d2 * lax.rsqrt(d2)) * INV_SIGMA, 0.0)
            rows.append(jnp.sum(w * xg, axis=0, keepdims=True))
            rows.append(jnp.sum(w * yg, axis=0, keepdims=True))
            rows.append(jnp.sum(w * zg, axis=0, keepdims=True))
        h = jnp.concatenate(rows, axis=0)
        out_ref[...] = lax.dot_general(
            h, w_ref[...], (((0,), (0,)), ((), ())),
            preferred_element_type=jnp.float32)

    return pl.pallas_call(
        body,
        grid=(nblk,),
        in_specs=[
            pl.BlockSpec((3, s, ql), lambda i: (0, 0, i)),
            pl.BlockSpec((3, ql), lambda i: (0, i)),
            pl.BlockSpec(memory_space=pltpu.SMEM),
            pl.BlockSpec((45, 64), lambda i: (0, 0)),
        ],
        out_specs=pl.BlockSpec((ql, 64), lambda i: (i, 0)),
        out_shape=jax.ShapeDtypeStruct((qpad, 64), jnp.float32),
    )(gath, centT, kpT, w45)


def _tc_bn(x, gamma1, beta1, q):
    def body(x_ref, g_ref, b_ref, o_ref):
        xv = x_ref[...]
        rid = lax.broadcasted_iota(jnp.int32, xv.shape, 0)
        m = rid < q
        xz = jnp.where(m, xv, 0.0)
        mean = jnp.sum(xz, axis=0, keepdims=True) * (1.0 / q)
        xc = xv - mean
        var = jnp.sum(jnp.where(m, xc * xc, 0.0), axis=0, keepdims=True) * (1.0 / q)
        y = xc * lax.rsqrt(var + 1e-5) * g_ref[...] + b_ref[...]
        o_ref[...] = jnp.where(y >= 0, y, 0.2 * y)[:q]

    return pl.pallas_call(
        body,
        out_shape=jax.ShapeDtypeStruct((q, 64), jnp.float32),
    )(x, gamma1, beta1)


def kernel(xyz, centors, idx, K_points, weight, gamma, beta):
    b, n, _ = xyz.shape
    num_group = centors.shape[1]
    q = b * num_group
    s = idx.shape[0] // q
    qpad = ((q + 1023) // 1024) * 1024

    xyzT_flat = xyz.reshape(n, 3).T.reshape(-1)
    idx_t = idx.reshape(q, s).T.reshape(-1)
    centT = jnp.pad(centors.reshape(q, 3).T, ((0, 0), (0, qpad - q)))
    kpT = K_points.T
    w45 = weight[:, 3:6, :].reshape(45, 64)

    gathered = _sc_gather(xyzT_flat, idx_t, n, q, s, qpad)
    gath = gathered.reshape(3, s, qpad)
    out_pre = _tc_conv(gath, centT, kpT, w45, qpad, s, ql=1024)
    out = _tc_bn(out_pre, gamma.reshape(1, 64), beta.reshape(1, 64), q)
    return out.reshape(b, num_group, 64)


# speedup vs baseline: 44.2155x; 1.0052x over previous
import functools

import jax
import jax.numpy as jnp
from jax import lax
from jax.experimental import pallas as pl
from jax.experimental.pallas import tpu as pltpu
from jax.experimental.pallas import tpu_sc as plsc

POINT_INFLUENCE = 0.04 * 30.0
INV_SIGMA = 1.0 / POINT_INFLUENCE

NC = 2
NS = 16
NW = NC * NS


def _sc_gather(xyzT_flat, idx_t, n, q, s, q0, h1, h2, stride):
    assert h2 % 16 == 0 and h1 % 16 == 0 and n % 8 == 0
    assert q0 % 8 == 0 and h1 % 8 == 0 and stride % 8 == 0
    max_rows = (s + 9) // 10
    nslot = 2 * max_rows

    mesh = plsc.VectorSubcoreMesh(core_axis_name="c", subcore_axis_name="s")

    @functools.partial(
        pl.kernel,
        out_type=jax.ShapeDtypeStruct((3 * s * stride,), jnp.float32),
        mesh=mesh,
        scratch_types=[
            pltpu.VMEM((n,), jnp.float32),
            pltpu.VMEM((h1,), jnp.int32),
            pltpu.VMEM((h2,), jnp.int32),
            pltpu.VMEM((h1,), jnp.float32),
            pltpu.VMEM((h2,), jnp.float32),
            pltpu.SemaphoreType.DMA,
            pltpu.SemaphoreType.DMA,
            pltpu.SemaphoreType.DMA,
            pltpu.SemaphoreType.DMA,
            pltpu.SemaphoreType.DMA,
        ],
        compiler_params=pltpu.CompilerParams(needs_layout_passes=False),
    )
    def sc_kernel(xyz_hbm, idx_hbm, out_hbm, plane_v, idx_v0, idx_v1,
                  val_v0, val_v1, sem_p, sem_i0, sem_i1, sem_o0, sem_o1):
        wid = lax.axis_index("s") * NC + lax.axis_index("c")
        coord = jnp.where(wid < 11, 0, jnp.where(wid < 22, 1, 2))
        base = wid - jnp.where(wid < 11, 0, jnp.where(wid < 22, 11, 22))
        tiles = jnp.where(wid < 22, 11, 10)
        r0 = base * s // tiles
        r1 = (base + 1) * s // tiles
        nh = 2 * (r1 - r0)

        sem_i = (sem_i0, sem_i1)
        sem_o = (sem_o0, sem_o1)
        idx_bufs = (idx_v0, idx_v1)
        val_bufs = (val_v0, val_v1)

        def task_offs(h):
            j = r0 + h // 2
            half = h % 2
            ioff = j * q + q0 + half * h1
            ooff = (coord * s + j) * stride + half * h1
            ln = h1 if half == 0 else h2
            return ioff, ooff, ln

        def idx_cp(h):
            ioff, _, ln = task_offs(h)
            return pltpu.make_async_copy(
                idx_hbm.at[pl.ds(ioff, ln)], idx_bufs[h % 2], sem_i[h % 2])

        def out_cp(h):
            _, ooff, ln = task_offs(h)
            return pltpu.make_async_copy(
                val_bufs[h % 2], out_hbm.at[pl.ds(ooff, ln)], sem_o[h % 2])

        plane_cp = pltpu.make_async_copy(
            xyz_hbm.at[pl.ds(coord * n, n)], plane_v, sem_p)
        plane_cp.start()
        idx_cp(0).start()
        idx_cp(1).start()
        plane_cp.wait()

        for h in range(nslot):
            b = h % 2
            if h >= 2:
                @pl.when(h - 2 < nh)
                def _(h=h):
                    out_cp(h - 2).wait()

            @pl.when(h < nh)
            def _(h=h, b=b):
                idx_cp(h).wait()
                _, _, ln = task_offs(h)

                @plsc.parallel_loop(0, ln, 16, unroll=8)
                def _(u):
                    iv = idx_bufs[b][pl.ds(u, 16)]
                    val_bufs[b][pl.ds(u, 16)] = plsc.load_gather(
                        plane_v, [iv])
                out_cp(h).start()

            if h + 2 < nslot:
                @pl.when(h + 2 < nh)
                def _(h=h):
                    idx_cp(h + 2).start()

        for h in range(nslot - 2, nslot):
            @pl.when(h < nh)
            def _(h=h):
                out_cp(h).wait()

    return sc_kernel(xyzT_flat, idx_t)


def _tc_conv(gath, centT, kpT, w45, qpad, s, ql):
    nblk = qpad // ql

    def body(gath_ref, cent_ref, kp_ref, w_ref, out_ref):
        xg = gath_ref[0]
        yg = gath_ref[1]
        zg = gath_ref[2]
        cx = cent_ref[0:1, :]
        cy = cent_ref[1:2, :]
        cz = cent_ref[2:3, :]
        relx = xg - cx
        rely = yg - cy
        relz = zg - cz
        rows = []
        for k in range(15):
            dx = relx - kp_ref[0, k]
            dy = rely - kp_ref[1, k]
            dz = relz - kp_ref[2, k]
            d2 = jnp.maximum(dx * dx + dy * dy + dz * dz, 1e-24)
            w = jnp.maximum(1.0 - (d2 * lax.rsqrt(d2)) * INV_SIGMA, 0.0)
            rows.append(jnp.sum(w * xg, axis=0, keepdims=True))
            rows.append(jnp.sum(w * yg, axis=0, keepdims=True))
            rows.append(jnp.sum(w * zg, axis=0, keepdims=True))
        h = jnp.concatenate(rows, axis=0)
        out_ref[...] = lax.dot_general(
            h, w_ref[...], (((0,), (0,)), ((), ())),
            preferred_element_type=jnp.float32)

    return pl.pallas_call(
        body,
        grid=(nblk,),
        in_specs=[
            pl.BlockSpec((3, s, ql), lambda i: (0, 0, i)),
            pl.BlockSpec((3, ql), lambda i: (0, i)),
            pl.BlockSpec(memory_space=pltpu.SMEM),
            pl.BlockSpec((45, 64), lambda i: (0, 0)),
        ],
        out_specs=pl.BlockSpec((ql, 64), lambda i: (i, 0)),
        out_shape=jax.ShapeDtypeStruct((qpad, 64), jnp.float32),
    )(gath, centT, kpT, w45)


def _tc_bn(x0, x1, gamma1, beta1, q, v1):
    n0 = x0.shape[0]

    def body(x0_ref, x1_ref, g_ref, b_ref, o_ref):
        v0 = x0_ref[...]
        w1 = x1_ref[...]
        rid = lax.broadcasted_iota(jnp.int32, w1.shape, 0)
        m = rid < v1
        mean = (jnp.sum(v0, axis=0, keepdims=True)
                + jnp.sum(jnp.where(m, w1, 0.0), axis=0, keepdims=True)) * (1.0 / q)
        c0 = v0 - mean
        c1 = w1 - mean
        var = (jnp.sum(c0 * c0, axis=0, keepdims=True)
               + jnp.sum(jnp.where(m, c1 * c1, 0.0), axis=0, keepdims=True)) * (1.0 / q)
        scale = lax.rsqrt(var + 1e-5) * g_ref[...]
        y0 = c0 * scale + b_ref[...]
        y1 = c1 * scale + b_ref[...]
        o_ref[0:n0, :] = jnp.where(y0 >= 0, y0, 0.2 * y0)
        o_ref[n0 : n0 + v1, :] = jnp.where(y1 >= 0, y1, 0.2 * y1)[:v1]

    return pl.pallas_call(
        body,
        out_shape=jax.ShapeDtypeStruct((q, 64), jnp.float32),
    )(x0, x1, gamma1, beta1)


def kernel(xyz, centors, idx, K_points, weight, gamma, beta):
    b, n, _ = xyz.shape
    num_group = centors.shape[1]
    q = b * num_group
    s = idx.shape[0] // q
    half = 5120

    xyzT_flat = xyz.reshape(n, 3).T.reshape(-1)
    idx_t = idx.reshape(q, s).T.reshape(-1)
    centT = jnp.pad(centors.reshape(q, 3).T, ((0, 0), (0, 2 * half - q)))
    kpT = K_points.T
    w45 = weight[:, 3:6, :].reshape(45, 64)

    g0 = _sc_gather(xyzT_flat, idx_t, n, q, s, 0, 2560, 2560, half)
    g1 = _sc_gather(xyzT_flat, idx_t, n, q, s, half, 2448, 2432, half)
    c0 = _tc_conv(g0.reshape(3, s, half), centT[:, :half], kpT, w45,
                  half, s, ql=1024)
    c1 = _tc_conv(g1.reshape(3, s, half), centT[:, half:], kpT, w45,
                  half, s, ql=1024)
    out = _tc_bn(c0, c1, gamma.reshape(1, 64), beta.reshape(1, 64),
                 q, q - half)
    return out.reshape(b, num_group, 64)


# speedup vs baseline: 44.3281x; 1.0025x over previous
import functools

import jax
import jax.numpy as jnp
from jax import lax
from jax.experimental import pallas as pl
from jax.experimental.pallas import tpu as pltpu
from jax.experimental.pallas import tpu_sc as plsc

POINT_INFLUENCE = 0.04 * 30.0
INV_SIGMA = 1.0 / POINT_INFLUENCE

NC = 2
NS = 16
NW = NC * NS


def _sc_gather(xyzT_flat, idx_t, n, q, s, q0, h1, h2, stride):
    assert h2 % 16 == 0 and h1 % 16 == 0 and n % 8 == 0
    assert q0 % 8 == 0 and h1 % 8 == 0 and stride % 8 == 0
    max_rows = (s + 9) // 10
    nslot = 2 * max_rows

    mesh = plsc.VectorSubcoreMesh(core_axis_name="c", subcore_axis_name="s")

    @functools.partial(
        pl.kernel,
        out_type=jax.ShapeDtypeStruct((3 * s * stride,), jnp.float32),
        mesh=mesh,
        scratch_types=[
            pltpu.VMEM((n,), jnp.float32),
            pltpu.VMEM((h1,), jnp.int32),
            pltpu.VMEM((h2,), jnp.int32),
            pltpu.VMEM((h1,), jnp.float32),
            pltpu.VMEM((h2,), jnp.float32),
            pltpu.SemaphoreType.DMA,
            pltpu.SemaphoreType.DMA,
            pltpu.SemaphoreType.DMA,
            pltpu.SemaphoreType.DMA,
            pltpu.SemaphoreType.DMA,
        ],
        compiler_params=pltpu.CompilerParams(needs_layout_passes=False),
    )
    def sc_kernel(xyz_hbm, idx_hbm, out_hbm, plane_v, idx_v0, idx_v1,
                  val_v0, val_v1, sem_p, sem_i0, sem_i1, sem_o0, sem_o1):
        wid = lax.axis_index("s") * NC + lax.axis_index("c")
        coord = jnp.where(wid < 11, 0, jnp.where(wid < 22, 1, 2))
        base = wid - jnp.where(wid < 11, 0, jnp.where(wid < 22, 11, 22))
        tiles = jnp.where(wid < 22, 11, 10)
        r0 = base * s // tiles
        r1 = (base + 1) * s // tiles
        nh = 2 * (r1 - r0)

        sem_i = (sem_i0, sem_i1)
        sem_o = (sem_o0, sem_o1)
        idx_bufs = (idx_v0, idx_v1)
        val_bufs = (val_v0, val_v1)

        def task_offs(h):
            j = r0 + h // 2
            half = h % 2
            ioff = j * q + q0 + half * h1
            ooff = (coord * s + j) * stride + half * h1
            ln = h1 if half == 0 else h2
            return ioff, ooff, ln

        def idx_cp(h):
            ioff, _, ln = task_offs(h)
            return pltpu.make_async_copy(
                idx_hbm.at[pl.ds(ioff, ln)], idx_bufs[h % 2], sem_i[h % 2])

        def out_cp(h):
            _, ooff, ln = task_offs(h)
            return pltpu.make_async_copy(
                val_bufs[h % 2], out_hbm.at[pl.ds(ooff, ln)], sem_o[h % 2])

        plane_cp = pltpu.make_async_copy(
            xyz_hbm.at[pl.ds(coord * n, n)], plane_v, sem_p)
        plane_cp.start()
        idx_cp(0).start()
        idx_cp(1).start()
        plane_cp.wait()

        for h in range(nslot):
            b = h % 2
            if h >= 2:
                @pl.when(h - 2 < nh)
                def _(h=h):
                    out_cp(h - 2).wait()

            @pl.when(h < nh)
            def _(h=h, b=b):
                idx_cp(h).wait()
                _, _, ln = task_offs(h)

                @plsc.parallel_loop(0, ln, 16, unroll=8)
                def _(u):
                    iv = idx_bufs[b][pl.ds(u, 16)]
                    val_bufs[b][pl.ds(u, 16)] = plsc.load_gather(
                        plane_v, [iv])
                out_cp(h).start()

            if h + 2 < nslot:
                @pl.when(h + 2 < nh)
                def _(h=h):
                    idx_cp(h + 2).start()

        for h in range(nslot - 2, nslot):
            @pl.when(h < nh)
            def _(h=h):
                out_cp(h).wait()

    return sc_kernel(xyzT_flat, idx_t)


def _tc_conv(gath, centT, kpT, w45, qpad, s, ql):
    nblk = qpad // ql

    def body(gath_ref, cent_ref, kp_ref, w_ref, out_ref):
        xg = gath_ref[0]
        yg = gath_ref[1]
        zg = gath_ref[2]
        cx = cent_ref[0:1, :]
        cy = cent_ref[1:2, :]
        cz = cent_ref[2:3, :]
        relx = xg - cx
        rely = yg - cy
        relz = zg - cz
        rows = []
        for k in range(15):
            dx = relx - kp_ref[0, k]
            dy = rely - kp_ref[1, k]
            dz = relz - kp_ref[2, k]
            d2 = jnp.maximum(dx * dx + dy * dy + dz * dz, 1e-24)
            w = jnp.maximum(1.0 - (d2 * lax.rsqrt(d2)) * INV_SIGMA, 0.0)
            rows.append(jnp.sum(w * xg, axis=0, keepdims=True))
            rows.append(jnp.sum(w * yg, axis=0, keepdims=True))
            rows.append(jnp.sum(w * zg, axis=0, keepdims=True))
        h = jnp.concatenate(rows, axis=0)
        out_ref[...] = lax.dot_general(
            h, w_ref[...], (((0,), (0,)), ((), ())),
            preferred_element_type=jnp.float32)

    return pl.pallas_call(
        body,
        grid=(nblk,),
        in_specs=[
            pl.BlockSpec((3, s, ql), lambda i: (0, 0, i)),
            pl.BlockSpec((3, ql), lambda i: (0, i)),
            pl.BlockSpec(memory_space=pltpu.SMEM),
            pl.BlockSpec((45, 64), lambda i: (0, 0)),
        ],
        out_specs=pl.BlockSpec((ql, 64), lambda i: (i, 0)),
        out_shape=jax.ShapeDtypeStruct((qpad, 64), jnp.float32),
    )(gath, centT, kpT, w45)


def _tc_bn(x0, x1, gamma1, beta1, q, v1):
    n0 = x0.shape[0]

    def body(x0_ref, x1_ref, g_ref, b_ref, o_ref):
        v0 = x0_ref[...]
        w1 = x1_ref[...]
        rid = lax.broadcasted_iota(jnp.int32, w1.shape, 0)
        m = rid < v1
        mean = (jnp.sum(v0, axis=0, keepdims=True)
                + jnp.sum(jnp.where(m, w1, 0.0), axis=0, keepdims=True)) * (1.0 / q)
        c0 = v0 - mean
        c1 = w1 - mean
        var = (jnp.sum(c0 * c0, axis=0, keepdims=True)
               + jnp.sum(jnp.where(m, c1 * c1, 0.0), axis=0, keepdims=True)) * (1.0 / q)
        scale = lax.rsqrt(var + 1e-5) * g_ref[...]
        y0 = c0 * scale + b_ref[...]
        y1 = c1 * scale + b_ref[...]
        o_ref[0, 0:n0, :] = jnp.where(y0 >= 0, y0, 0.2 * y0)
        o_ref[0, n0 : n0 + v1, :] = jnp.where(y1 >= 0, y1, 0.2 * y1)[:v1]

    return pl.pallas_call(
        body,
        out_shape=jax.ShapeDtypeStruct((1, q, 64), jnp.float32),
    )(x0, x1, gamma1, beta1)


def kernel(xyz, centors, idx, K_points, weight, gamma, beta):
    b, n, _ = xyz.shape
    num_group = centors.shape[1]
    q = b * num_group
    s = idx.shape[0] // q
    half = 5120

    xyzT_flat = xyz.reshape(n, 3).T.reshape(-1)
    idx_t = idx.reshape(q, s).T.reshape(-1)
    centT = jnp.pad(centors.reshape(q, 3).T, ((0, 0), (0, 2 * half - q)))
    kpT = K_points.T
    w45 = weight[:, 3:6, :].reshape(45, 64)

    g0 = _sc_gather(xyzT_flat, idx_t, n, q, s, 0, 2560, 2560, half)
    g1 = _sc_gather(xyzT_flat, idx_t, n, q, s, half, 2448, 2432, half)
    c0 = _tc_conv(g0.reshape(3, s, half), centT[:, :half], kpT, w45,
                  half, s, ql=1024)
    c1 = _tc_conv(g1.reshape(3, s, half), centT[:, half:], kpT, w45,
                  half, s, ql=1024)
    return _tc_bn(c0, c1, gamma.reshape(1, 64), beta.reshape(1, 64),
                  q, q - half)
